# baseline (speedup 1.0000x reference)
import functools

import jax
import jax.numpy as jnp
from jax import lax
from jax.experimental import pallas as pl
from jax.experimental.pallas import tpu as pltpu
from jax.experimental.pallas import tpu_sc as plsc

N = 100000
B = 32768
D_MEM = 128
D_MSG = 16
D_TIME = 32
D_OUT = 128

_BLK = 2048


def _dense_body(mem_src_ref, mem_dst_ref, msg_ref, t_ref, lu_src_ref,
                w_t_ref, b_t_ref, W_nbr_ref, W_upd_ref,
                m_ref, upd_ref):
    mem_src = mem_src_ref[...]
    mem_dst = mem_dst_ref[...]
    msg = msg_ref[...]
    rel = lu_src_ref[...] - t_ref[...]
    w_t = w_t_ref[...]
    b_t = b_t_ref[...]
    enc_rel = jnp.cos(rel * w_t + b_t)
    enc_t = jnp.cos((-rel) * w_t + b_t)
    nbr_in = jnp.concatenate([mem_src, msg, enc_rel], axis=1)
    m_ref[...] = jax.nn.relu(
        jax.lax.dot(nbr_in, W_nbr_ref[...],
                    preferred_element_type=jnp.float32))
    upd_in = jnp.concatenate([mem_src, mem_dst, msg, enc_t], axis=1)
    upd_ref[...] = jnp.tanh(
        jax.lax.dot(upd_in, W_upd_ref[...],
                    preferred_element_type=jnp.float32))


def _dense(mem_src, mem_dst, msg, t, lu_src, w_t, b_t, W_nbr, W_upd):
    grid = (B // _BLK,)
    row_spec = lambda d: pl.BlockSpec((_BLK, d), lambda i: (i, 0))
    full = lambda a, b: pl.BlockSpec((a, b), lambda i: (0, 0))
    return pl.pallas_call(
        _dense_body,
        grid=grid,
        in_specs=[
            row_spec(D_MEM), row_spec(D_MEM), row_spec(D_MSG),
            row_spec(1), row_spec(1),
            full(1, D_TIME), full(1, D_TIME),
            full(D_MEM + D_MSG + D_TIME, D_OUT),
            full(2 * D_MEM + D_MSG + D_TIME, D_MEM),
        ],
        out_specs=[row_spec(D_OUT), row_spec(D_MEM)],
        out_shape=[
            jax.ShapeDtypeStruct((B, D_OUT), jnp.float32),
            jax.ShapeDtypeStruct((B, D_MEM), jnp.float32),
        ],
    )(mem_src, mem_dst, msg, t, lu_src, w_t, b_t, W_nbr, W_upd)


def _head_body(mem_src_ref, mem_dst_ref, as_ref, ad_ref, W_self_ref,
               hs_ref, hd_ref):
    W_self = W_self_ref[...]
    hs_ref[...] = jax.nn.relu(
        jax.lax.dot(mem_src_ref[...], W_self,
                    preferred_element_type=jnp.float32) + as_ref[...])
    hd_ref[...] = jax.nn.relu(
        jax.lax.dot(mem_dst_ref[...], W_self,
                    preferred_element_type=jnp.float32) + ad_ref[...])


def _head(mem_src, mem_dst, agg_src, agg_dst, W_self):
    grid = (B // _BLK,)
    row_spec = lambda d: pl.BlockSpec((_BLK, d), lambda i: (i, 0))
    full = lambda a, b: pl.BlockSpec((a, b), lambda i: (0, 0))
    return pl.pallas_call(
        _head_body,
        grid=grid,
        in_specs=[row_spec(D_MEM), row_spec(D_MEM),
                  row_spec(D_OUT), row_spec(D_OUT),
                  full(D_MEM, D_OUT)],
        out_specs=[row_spec(D_OUT), row_spec(D_OUT)],
        out_shape=[
            jax.ShapeDtypeStruct((B, D_OUT), jnp.float32),
            jax.ShapeDtypeStruct((B, D_OUT), jnp.float32),
        ],
    )(mem_src, mem_dst, agg_src, agg_dst, W_self)


_INFO = plsc.get_sparse_core_info()
_NC, _NS = _INFO.num_cores, _INFO.num_subcores
_NW = _NC * _NS
_EV_W = B // _NW
_CH = 512


def _gather_body(mem_hbm, ei_hbm, lu_hbm, ms_out, md_out, lus_out,
                 idx_v, rows_v, lu_v, sem):
    wid = lax.axis_index("s") * _NC + lax.axis_index("c")
    base = wid * _EV_W
    pltpu.sync_copy(ei_hbm.at[0, pl.ds(base, _EV_W)], idx_v)
    pltpu.async_copy(lu_hbm.at[idx_v], lu_v, sem).wait()
    pltpu.sync_copy(lu_v, lus_out.at[pl.ds(base, _EV_W)])
    for half in range(_EV_W // _CH):
        off = half * _CH
        pltpu.async_copy(mem_hbm.at[idx_v.at[pl.ds(off, _CH)]], rows_v,
                         sem).wait()
        pltpu.sync_copy(rows_v, ms_out.at[pl.ds(base + off, _CH)])
    pltpu.sync_copy(ei_hbm.at[1, pl.ds(base, _EV_W)], idx_v)
    for half in range(_EV_W // _CH):
        off = half * _CH
        pltpu.async_copy(mem_hbm.at[idx_v.at[pl.ds(off, _CH)]], rows_v,
                         sem).wait()
        pltpu.sync_copy(rows_v, md_out.at[pl.ds(base + off, _CH)])


_sc_gather = pl.kernel(
    _gather_body,
    out_type=[
        jax.ShapeDtypeStruct((B, D_MEM), jnp.float32),
        jax.ShapeDtypeStruct((B, D_MEM), jnp.float32),
        jax.ShapeDtypeStruct((B,), jnp.float32),
    ],
    mesh=plsc.VectorSubcoreMesh(core_axis_name="c", subcore_axis_name="s"),
    scratch_types=[
        pltpu.VMEM((_EV_W,), jnp.int32),
        pltpu.VMEM((_CH, D_MEM), jnp.float32),
        pltpu.VMEM((_EV_W,), jnp.float32),
        pltpu.SemaphoreType.DMA,
    ],
)


_AGG_R = 13055
_AGG_PASSES = (N + 2 * _AGG_R - 1) // (2 * _AGG_R)
_EV_T = B // _NS
_ACH = 64
_CROWS = (_EV_T + _ACH - 1) // _ACH + 1
_ZSH = (_AGG_R + 1) // _NS
_ZCH = 48


def _agg_compact(ev_ref, base, iota, cv, cb, sid):
    def compact(j, off):
        v = ev_ref[pl.ds(j * 16, 16)]
        rel = v - base
        m_in = (rel >= 0) & (rel < _AGG_R)
        bv = sid * _EV_T + j * 16 + iota
        plsc.store_compressed(cv.at[pl.ds(off, 16)], rel, mask=m_in)
        plsc.store_compressed(cb.at[pl.ds(off, 16)], bv, mask=m_in)
        return off + jnp.sum(m_in.astype(jnp.int32))
    return lax.fori_loop(0, _EV_T // 16, compact, jnp.int32(0))


def _agg_pad(cv, cb, off, iota, padv, padb):
    nch = (off + _ACH - 1) // _ACH

    def pad(j, _):
        mval = (j * 16 + iota) < off
        cv[pl.ds(j * 16, 16)] = jnp.where(mval, cv[pl.ds(j * 16, 16)], padv)
        cb[pl.ds(j * 16, 16)] = jnp.where(mval, cb[pl.ds(j * 16, 16)], padb)
        return 0
    lax.fori_loop(off // 16, nch * (_ACH // 16), pad, 0)
    return nch


def _agg_tocol(cv, cv2, nch):
    vpr = _ACH // 16

    def tocol(j, _):
        cv2[j // vpr, pl.ds((j % vpr) * 16, 16)] = cv[pl.ds(j * 16, 16)]
        return 0
    lax.fori_loop(0, nch * vpr, tocol, 0)


def _agg_body(ei_hbm, m_hbm, as_hbm, ad_hbm,
              ev_src, ev_dst, cvd, cbd, cvs, cbs, cv2, rows,
              shared, sem):
    sid = lax.axis_index("s")
    core = lax.axis_index("c")
    iota = lax.iota(jnp.int32, 16)
    z16 = jnp.zeros((16,), jnp.float32)
    pltpu.sync_copy(ei_hbm.at[0, pl.ds(sid * _EV_T, _EV_T)], ev_src)
    pltpu.sync_copy(ei_hbm.at[1, pl.ds(sid * _EV_T, _EV_T)], ev_dst)

    def one_pass(r, _):
        base = r * (2 * _AGG_R) + core * _AGG_R

        def zb(i, _):
            rows[i // 8, pl.ds((i % 8) * 16, 16)] = z16
            return 0
        lax.fori_loop(0, _ACH * 8, zb, 0)

        def zero(q, _):
            pltpu.sync_copy(rows.at[pl.ds(0, _ZCH)],
                            shared.at[pl.ds(sid * _ZSH + q * _ZCH, _ZCH)])
            return 0
        lax.fori_loop(0, _ZSH // _ZCH, zero, 0)
        plsc.subcore_barrier()

        offd = _agg_compact(ev_dst, base, iota, cvd, cbd, sid)
        nchd = _agg_pad(cvd, cbd, offd, iota,
                        jnp.zeros((16,), jnp.int32) + _AGG_R,
                        jnp.zeros((16,), jnp.int32))
        _agg_tocol(cvd, cv2, nchd)

        def add_chunk(c, _):
            pltpu.async_copy(m_hbm.at[cbd.at[pl.ds(c * _ACH, _ACH)]], rows,
                             sem).wait()
            pltpu.sync_copy(rows, shared.at[cv2.at[c]], add=True)
            return 0
        lax.fori_loop(0, nchd, add_chunk, 0)
        plsc.subcore_barrier()

        def emit(cv, cb, off, out_hbm):
            @pl.when(off > 0)
            def _():
                lastv = plsc.load_gather(cv, [iota * 0 + off - 1])
                lastb = plsc.load_gather(cb, [iota * 0 + off - 1])
                nch = _agg_pad(cv, cb, off, iota, lastv, lastb)
                _agg_tocol(cb, cv2, nch)

                def out_chunk(c, _):
                    pltpu.async_copy(
                        shared.at[cv.at[pl.ds(c * _ACH, _ACH)]], rows,
                        sem).wait()
                    pltpu.sync_copy(rows, out_hbm.at[cv2.at[c]])
                    return 0
                lax.fori_loop(0, nch, out_chunk, 0)

        offs = _agg_compact(ev_src, base, iota, cvs, cbs, sid)
        emit(cvs, cbs, offs, as_hbm)
        emit(cvd, cbd, offd, ad_hbm)
        plsc.subcore_barrier()
        return 0
    lax.fori_loop(0, _AGG_PASSES, one_pass, 0)


_sc_agg = pl.kernel(
    _agg_body,
    out_type=[
        jax.ShapeDtypeStruct((B, D_OUT), jnp.float32),
        jax.ShapeDtypeStruct((B, D_OUT), jnp.float32),
    ],
    mesh=plsc.VectorSubcoreMesh(core_axis_name="c", subcore_axis_name="s"),
    scratch_types=[
        pltpu.VMEM((_EV_T,), jnp.int32),
        pltpu.VMEM((_EV_T,), jnp.int32),
        pltpu.VMEM((_EV_T + 16,), jnp.int32),
        pltpu.VMEM((_EV_T + 16,), jnp.int32),
        pltpu.VMEM((_EV_T + 16,), jnp.int32),
        pltpu.VMEM((_EV_T + 16,), jnp.int32),
        pltpu.VMEM((_CROWS, _ACH), jnp.int32),
        pltpu.VMEM((_ACH, D_OUT), jnp.float32),
        pltpu.VMEM_SHARED((_AGG_R + 1, D_OUT), jnp.float32),
        pltpu.SemaphoreType.DMA,
    ],
    compiler_params=pltpu.CompilerParams(needs_layout_passes=False),
)


_CHK = 128
_OWN = 3200
_LAST = N - (_NW - 1) * _OWN
_PCAP = _OWN
_STRIP = 2048
_NSTRIP = B // _STRIP


def _update_body(mem_hbm, ei_hbm, t_hbm, lu_hbm, upd_hbm,
                 newmem_hbm, newlu_hbm,
                 ss, cv, cb, pos, wn1, wp1, wn2, wrows, luv, twv, sem, semc):
    wid = lax.axis_index("s") * _NC + lax.axis_index("c")
    base = wid * _OWN
    iota = lax.iota(jnp.int32, 16)
    neg1 = jnp.zeros((16,), jnp.int32) - 1

    @pl.when(wid < _NW - 1)
    def _():
        pltpu.async_copy(mem_hbm.at[pl.ds(base, _OWN)],
                         newmem_hbm.at[pl.ds(base, _OWN)], semc)

    @pl.when(wid == _NW - 1)
    def _():
        pltpu.async_copy(mem_hbm.at[pl.ds((_NW - 1) * _OWN, _LAST)],
                         newmem_hbm.at[pl.ds((_NW - 1) * _OWN, _LAST)], semc)

    def p1(i, _):
        pos[pl.ds(i * 16, 16)] = neg1
        return 0
    lax.fori_loop(0, _PCAP // 16, p1, 0)

    def p2(s, _):
        pltpu.sync_copy(ei_hbm.at[0, pl.ds(s * _STRIP, _STRIP)], ss)

        def compact(j, off):
            v = ss[pl.ds(j * 16, 16)]
            slot = v - base
            own = (slot >= 0) & (slot < _OWN)
            bv = s * _STRIP + j * 16 + iota
            plsc.store_compressed(cv.at[pl.ds(off, 16)], slot, mask=own)
            plsc.store_compressed(cb.at[pl.ds(off, 16)], bv, mask=own)
            return off + jnp.sum(own.astype(jnp.int32))
        off = lax.fori_loop(0, _STRIP // 16, compact, jnp.int32(0))

        def scat(k, _):
            sl = cv[pl.ds(k * 16, 16)]
            bvv = cb[pl.ds(k * 16, 16)]
            valid = (k * 16 + iota) < off
            _, lastm = plsc.scan_count(sl, valid)
            plsc.store_scatter(pos, [sl], bvv, mask=lastm & valid)
            return 0
        lax.fori_loop(0, (off + 15) // 16, scat, 0)
        return 0
    lax.fori_loop(0, _NSTRIP, p2, 0)

    def p3(j, w):
        p = pos[pl.ds(j * 16, 16)]
        m = p >= 0
        node = base + j * 16 + iota
        plsc.store_compressed(wn1.at[pl.ds(w, 16)], node, mask=m)
        plsc.store_compressed(wp1.at[pl.ds(w, 16)], p, mask=m)
        return w + jnp.sum(m.astype(jnp.int32))
    w_cnt = lax.fori_loop(0, _PCAP // 16, p3, jnp.int32(0))

    n_wchunk = (w_cnt + _CHK - 1) // _CHK

    @pl.when(w_cnt > 0)
    def _():
        z16 = jnp.zeros((16,), jnp.int32)
        padn = plsc.load_gather(wn1, [z16])
        padp = plsc.load_gather(wp1, [z16])

        def pad(j, _):
            mval = (j * 16 + iota) < w_cnt
            wn1[pl.ds(j * 16, 16)] = jnp.where(mval, wn1[pl.ds(j * 16, 16)],
                                               padn)
            wp1[pl.ds(j * 16, 16)] = jnp.where(mval, wp1[pl.ds(j * 16, 16)],
                                               padp)
            return 0
        lax.fori_loop(w_cnt // 16, n_wchunk * (_CHK // 16), pad, 0)

        def tocol(j, _):
            wn2[j // 8, pl.ds((j % 8) * 16, 16)] = wn1[pl.ds(j * 16, 16)]
            return 0
        lax.fori_loop(0, n_wchunk * (_CHK // 16), tocol, 0)

    @pl.when(wid < _NW - 1)
    def _():
        pltpu.sync_copy(lu_hbm.at[pl.ds(base, _OWN)], luv)
        pltpu.sync_copy(luv, newlu_hbm.at[pl.ds(base, _OWN)])
        pltpu.make_async_copy(mem_hbm.at[pl.ds(base, _OWN)],
                              newmem_hbm.at[pl.ds(base, _OWN)], semc).wait()

    @pl.when(wid == _NW - 1)
    def _():
        lb = (_NW - 1) * _OWN
        pltpu.sync_copy(lu_hbm.at[pl.ds(lb, _LAST)], luv.at[pl.ds(0, _LAST)])
        pltpu.sync_copy(luv.at[pl.ds(0, _LAST)], newlu_hbm.at[pl.ds(lb, _LAST)])
        pltpu.make_async_copy(mem_hbm.at[pl.ds(lb, _LAST)],
                              newmem_hbm.at[pl.ds(lb, _LAST)], semc).wait()

    def winner_chunk(c2, _):
        pltpu.async_copy(upd_hbm.at[wp1.at[pl.ds(c2 * _CHK, _CHK)]], wrows,
                         sem).wait()
        pltpu.sync_copy(wrows, newmem_hbm.at[wn2.at[c2]])
        pltpu.async_copy(t_hbm.at[wp1.at[pl.ds(c2 * _CHK, _CHK)]], twv,
                         sem).wait()
        pltpu.sync_copy(twv, newlu_hbm.at[wn2.at[c2]])
        return 0
    lax.fori_loop(0, n_wchunk, winner_chunk, 0)


_sc_update = pl.kernel(
    _update_body,
    out_type=[
        jax.ShapeDtypeStruct((N, D_MEM), jnp.float32),
        jax.ShapeDtypeStruct((N,), jnp.float32),
    ],
    mesh=plsc.VectorSubcoreMesh(core_axis_name="c", subcore_axis_name="s"),
    scratch_types=[
        pltpu.VMEM((_STRIP,), jnp.int32),
        pltpu.VMEM((_STRIP + 16,), jnp.int32),
        pltpu.VMEM((_STRIP + 16,), jnp.int32),
        pltpu.VMEM((_PCAP,), jnp.int32),
        pltpu.VMEM((_PCAP + 16,), jnp.int32),
        pltpu.VMEM((_PCAP + 16,), jnp.int32),
        pltpu.VMEM((_PCAP // _CHK, _CHK), jnp.int32),
        pltpu.VMEM((_CHK, D_MEM), jnp.float32),
        pltpu.VMEM((_OWN,), jnp.float32),
        pltpu.VMEM((_CHK,), jnp.float32),
        pltpu.SemaphoreType.DMA,
        pltpu.SemaphoreType.DMA,
    ],
    compiler_params=pltpu.CompilerParams(needs_layout_passes=False),
)


def kernel(edge_index, t, msg, memory, last_update, w_t, b_t, W_nbr, W_self, W_upd):
    src, dst = edge_index[0], edge_index[1]
    mem_src, mem_dst, lu_src = _sc_gather(memory, edge_index, last_update)
    t2 = t[:, None]
    lu2 = lu_src[:, None]
    m, upd = _dense(
        mem_src, mem_dst, msg, t2, lu2, w_t, b_t.reshape(1, D_TIME),
        W_nbr, W_upd)
    agg_src, agg_dst = _sc_agg(edge_index, m)
    new_memory, new_last_update = _sc_update(
        memory, edge_index, t, last_update, upd)
    h_src, h_dst = _head(mem_src, mem_dst, agg_src, agg_dst, W_self)
    return (h_src, h_dst, new_memory, new_last_update)


# speedup vs baseline: 3.1226x; 3.1226x over previous
import functools

import jax
import jax.numpy as jnp
from jax import lax
from jax.experimental import pallas as pl
from jax.experimental.pallas import tpu as pltpu
from jax.experimental.pallas import tpu_sc as plsc

N = 100000
B = 32768
D_MEM = 128
D_MSG = 16
D_TIME = 32
D_OUT = 128

_BLK = 2048


def _dense_body(mem_src_ref, mem_dst_ref, msg_ref, t_ref, lu_src_ref,
                w_t_ref, b_t_ref, W_nbr_ref, W_upd_ref,
                m_ref, upd_ref):
    mem_src = mem_src_ref[...]
    mem_dst = mem_dst_ref[...]
    msg = msg_ref[...]
    rel = lu_src_ref[...] - t_ref[...]
    w_t = w_t_ref[...]
    b_t = b_t_ref[...]
    enc_rel = jnp.cos(rel * w_t + b_t)
    enc_t = jnp.cos((-rel) * w_t + b_t)
    nbr_in = jnp.concatenate([mem_src, msg, enc_rel], axis=1)
    m_ref[...] = jax.nn.relu(
        jax.lax.dot(nbr_in, W_nbr_ref[...],
                    preferred_element_type=jnp.float32))
    upd_in = jnp.concatenate([mem_src, mem_dst, msg, enc_t], axis=1)
    upd_ref[...] = jnp.tanh(
        jax.lax.dot(upd_in, W_upd_ref[...],
                    preferred_element_type=jnp.float32))


def _dense(mem_src, mem_dst, msg, t, lu_src, w_t, b_t, W_nbr, W_upd):
    grid = (B // _BLK,)
    row_spec = lambda d: pl.BlockSpec((_BLK, d), lambda i: (i, 0))
    full = lambda a, b: pl.BlockSpec((a, b), lambda i: (0, 0))
    return pl.pallas_call(
        _dense_body,
        grid=grid,
        in_specs=[
            row_spec(D_MEM), row_spec(D_MEM), row_spec(D_MSG),
            row_spec(1), row_spec(1),
            full(1, D_TIME), full(1, D_TIME),
            full(D_MEM + D_MSG + D_TIME, D_OUT),
            full(2 * D_MEM + D_MSG + D_TIME, D_MEM),
        ],
        out_specs=[row_spec(D_OUT), row_spec(D_MEM)],
        out_shape=[
            jax.ShapeDtypeStruct((B, D_OUT), jnp.float32),
            jax.ShapeDtypeStruct((B, D_MEM), jnp.float32),
        ],
    )(mem_src, mem_dst, msg, t, lu_src, w_t, b_t, W_nbr, W_upd)


def _head_body(mem_src_ref, mem_dst_ref, as_ref, ad_ref, W_self_ref,
               hs_ref, hd_ref):
    W_self = W_self_ref[...]
    hs_ref[...] = jax.nn.relu(
        jax.lax.dot(mem_src_ref[...], W_self,
                    preferred_element_type=jnp.float32) + as_ref[...])
    hd_ref[...] = jax.nn.relu(
        jax.lax.dot(mem_dst_ref[...], W_self,
                    preferred_element_type=jnp.float32) + ad_ref[...])


def _head(mem_src, mem_dst, agg_src, agg_dst, W_self):
    grid = (B // _BLK,)
    row_spec = lambda d: pl.BlockSpec((_BLK, d), lambda i: (i, 0))
    full = lambda a, b: pl.BlockSpec((a, b), lambda i: (0, 0))
    return pl.pallas_call(
        _head_body,
        grid=grid,
        in_specs=[row_spec(D_MEM), row_spec(D_MEM),
                  row_spec(D_OUT), row_spec(D_OUT),
                  full(D_MEM, D_OUT)],
        out_specs=[row_spec(D_OUT), row_spec(D_OUT)],
        out_shape=[
            jax.ShapeDtypeStruct((B, D_OUT), jnp.float32),
            jax.ShapeDtypeStruct((B, D_OUT), jnp.float32),
        ],
    )(mem_src, mem_dst, agg_src, agg_dst, W_self)


_INFO = plsc.get_sparse_core_info()
_NC, _NS = _INFO.num_cores, _INFO.num_subcores
_NW = _NC * _NS
_EV_W = B // _NW
_CH = 512


def _gather_body(mem_hbm, ei_hbm, lu_hbm, ms_out, md_out, lus_out,
                 idx_v, rows_v, lu_v, sem):
    wid = lax.axis_index("s") * _NC + lax.axis_index("c")
    base = wid * _EV_W
    pltpu.sync_copy(ei_hbm.at[0, pl.ds(base, _EV_W)], idx_v)
    pltpu.async_copy(lu_hbm.at[idx_v], lu_v, sem).wait()
    pltpu.sync_copy(lu_v, lus_out.at[pl.ds(base, _EV_W)])
    for half in range(_EV_W // _CH):
        off = half * _CH
        pltpu.async_copy(mem_hbm.at[idx_v.at[pl.ds(off, _CH)]], rows_v,
                         sem).wait()
        pltpu.sync_copy(rows_v, ms_out.at[pl.ds(base + off, _CH)])
    pltpu.sync_copy(ei_hbm.at[1, pl.ds(base, _EV_W)], idx_v)
    for half in range(_EV_W // _CH):
        off = half * _CH
        pltpu.async_copy(mem_hbm.at[idx_v.at[pl.ds(off, _CH)]], rows_v,
                         sem).wait()
        pltpu.sync_copy(rows_v, md_out.at[pl.ds(base + off, _CH)])


_sc_gather = pl.kernel(
    _gather_body,
    out_type=[
        jax.ShapeDtypeStruct((B, D_MEM), jnp.float32),
        jax.ShapeDtypeStruct((B, D_MEM), jnp.float32),
        jax.ShapeDtypeStruct((B,), jnp.float32),
    ],
    mesh=plsc.VectorSubcoreMesh(core_axis_name="c", subcore_axis_name="s"),
    scratch_types=[
        pltpu.VMEM((_EV_W,), jnp.int32),
        pltpu.VMEM((_CH, D_MEM), jnp.float32),
        pltpu.VMEM((_EV_W,), jnp.float32),
        pltpu.SemaphoreType.DMA,
    ],
)


_AGG_R = 13055
_AGG_PASSES = (N + 2 * _AGG_R - 1) // (2 * _AGG_R)
_EV_T = B // _NS
_ACH = 64
_CROWS = (_EV_T + _ACH - 1) // _ACH + 1
_ZSH = (_AGG_R + 1) // _NS
_ZCH = 48


def _agg_compact(ev_ref, base, iota, cv, cb, sid):
    def compact(j, off):
        v = ev_ref[pl.ds(j * 16, 16)]
        rel = v - base
        m_in = (rel >= 0) & (rel < _AGG_R)
        bv = sid * _EV_T + j * 16 + iota
        plsc.store_compressed(cv.at[pl.ds(off, 16)], rel, mask=m_in)
        plsc.store_compressed(cb.at[pl.ds(off, 16)], bv, mask=m_in)
        return off + jnp.sum(m_in.astype(jnp.int32))
    return lax.fori_loop(0, _EV_T // 16, compact, jnp.int32(0))


def _agg_pad(cv, cb, off, iota, padv, padb):
    nch = (off + _ACH - 1) // _ACH

    def pad(j, _):
        mval = (j * 16 + iota) < off
        cv[pl.ds(j * 16, 16)] = jnp.where(mval, cv[pl.ds(j * 16, 16)], padv)
        cb[pl.ds(j * 16, 16)] = jnp.where(mval, cb[pl.ds(j * 16, 16)], padb)
        return 0
    lax.fori_loop(off // 16, nch * (_ACH // 16), pad, 0)
    return nch


def _agg_tocol(cv, cv2, nch):
    vpr = _ACH // 16

    def tocol(j, _):
        cv2[j // vpr, pl.ds((j % vpr) * 16, 16)] = cv[pl.ds(j * 16, 16)]
        return 0
    lax.fori_loop(0, nch * vpr, tocol, 0)


def _agg_body(ei_hbm, m_hbm, as_hbm, ad_hbm,
              ev_src, ev_dst, cvd, cbd, cvs, cbs, cv2, rows,
              shared, sem):
    sid = lax.axis_index("s")
    core = lax.axis_index("c")
    iota = lax.iota(jnp.int32, 16)
    z16 = jnp.zeros((16,), jnp.float32)
    pltpu.sync_copy(ei_hbm.at[0, pl.ds(sid * _EV_T, _EV_T)], ev_src)
    pltpu.sync_copy(ei_hbm.at[1, pl.ds(sid * _EV_T, _EV_T)], ev_dst)

    def one_pass(r, _):
        base = r * (2 * _AGG_R) + core * _AGG_R

        def zb(i, _):
            rows[i // 8, pl.ds((i % 8) * 16, 16)] = z16
            return 0
        lax.fori_loop(0, _ACH * 8, zb, 0)

        def zero(q, _):
            pltpu.sync_copy(rows.at[pl.ds(0, _ZCH)],
                            shared.at[pl.ds(sid * _ZSH + q * _ZCH, _ZCH)])
            return 0
        lax.fori_loop(0, _ZSH // _ZCH, zero, 0)
        plsc.subcore_barrier()

        offd = _agg_compact(ev_dst, base, iota, cvd, cbd, sid)
        nchd = _agg_pad(cvd, cbd, offd, iota,
                        jnp.zeros((16,), jnp.int32) + _AGG_R,
                        jnp.zeros((16,), jnp.int32))
        _agg_tocol(cvd, cv2, nchd)

        def add_chunk(c, _):
            pltpu.async_copy(m_hbm.at[cbd.at[pl.ds(c * _ACH, _ACH)]], rows,
                             sem).wait()
            pltpu.sync_copy(rows, shared.at[cv2.at[c]], add=True)
            return 0
        lax.fori_loop(0, nchd, add_chunk, 0)
        plsc.subcore_barrier()

        def emit(cv, cb, off, out_hbm):
            @pl.when(off > 0)
            def _():
                lastv = plsc.load_gather(cv, [iota * 0 + off - 1])
                lastb = plsc.load_gather(cb, [iota * 0 + off - 1])
                nch = _agg_pad(cv, cb, off, iota, lastv, lastb)
                _agg_tocol(cb, cv2, nch)

                def out_chunk(c, _):
                    pltpu.async_copy(
                        shared.at[cv.at[pl.ds(c * _ACH, _ACH)]], rows,
                        sem).wait()
                    pltpu.sync_copy(rows, out_hbm.at[cv2.at[c]])
                    return 0
                lax.fori_loop(0, nch, out_chunk, 0)

        offs = _agg_compact(ev_src, base, iota, cvs, cbs, sid)
        emit(cvs, cbs, offs, as_hbm)
        emit(cvd, cbd, offd, ad_hbm)
        plsc.subcore_barrier()
        return 0
    lax.fori_loop(0, _AGG_PASSES, one_pass, 0)


_sc_agg = pl.kernel(
    _agg_body,
    out_type=[
        jax.ShapeDtypeStruct((B, D_OUT), jnp.float32),
        jax.ShapeDtypeStruct((B, D_OUT), jnp.float32),
    ],
    mesh=plsc.VectorSubcoreMesh(core_axis_name="c", subcore_axis_name="s"),
    scratch_types=[
        pltpu.VMEM((_EV_T,), jnp.int32),
        pltpu.VMEM((_EV_T,), jnp.int32),
        pltpu.VMEM((_EV_T + 16,), jnp.int32),
        pltpu.VMEM((_EV_T + 16,), jnp.int32),
        pltpu.VMEM((_EV_T + 16,), jnp.int32),
        pltpu.VMEM((_EV_T + 16,), jnp.int32),
        pltpu.VMEM((_CROWS, _ACH), jnp.int32),
        pltpu.VMEM((_ACH, D_OUT), jnp.float32),
        pltpu.VMEM_SHARED((_AGG_R + 1, D_OUT), jnp.float32),
        pltpu.SemaphoreType.DMA,
    ],
    compiler_params=pltpu.CompilerParams(needs_layout_passes=False),
)


_CHK = 128
_OWN = 3200
_LAST = N - (_NW - 1) * _OWN
_PCAP = _OWN
_STRIP = 2048
_NSTRIP = B // _STRIP


def _update_body(mem_hbm, ei_hbm, t_hbm, lu_hbm, upd_hbm,
                 newmem_hbm, newlu_hbm,
                 ss, cv, cb, pos, wn1, wp1, wn2, wrows, bufa, bufb, luv, twv,
                 sem, seml, sems):
    wid = lax.axis_index("s") * _NC + lax.axis_index("c")
    base = wid * _OWN
    iota = lax.iota(jnp.int32, 16)
    neg1 = jnp.zeros((16,), jnp.int32) - 1

    nch = (jnp.minimum(jnp.int32(_OWN), jnp.int32(N) - base)) // _CHK
    pltpu.async_copy(mem_hbm.at[pl.ds(base, _CHK)], bufa, seml)

    def p1(i, _):
        pos[pl.ds(i * 16, 16)] = neg1
        return 0
    lax.fori_loop(0, _PCAP // 16, p1, 0)

    def p2(s, _):
        pltpu.sync_copy(ei_hbm.at[0, pl.ds(s * _STRIP, _STRIP)], ss)

        def compact(j, off):
            v = ss[pl.ds(j * 16, 16)]
            slot = v - base
            own = (slot >= 0) & (slot < _OWN)
            bv = s * _STRIP + j * 16 + iota
            plsc.store_compressed(cv.at[pl.ds(off, 16)], slot, mask=own)
            plsc.store_compressed(cb.at[pl.ds(off, 16)], bv, mask=own)
            return off + jnp.sum(own.astype(jnp.int32))
        off = lax.fori_loop(0, _STRIP // 16, compact, jnp.int32(0))

        def scat(k, _):
            sl = cv[pl.ds(k * 16, 16)]
            bvv = cb[pl.ds(k * 16, 16)]
            valid = (k * 16 + iota) < off
            _, lastm = plsc.scan_count(sl, valid)
            plsc.store_scatter(pos, [sl], bvv, mask=lastm & valid)
            return 0
        lax.fori_loop(0, (off + 15) // 16, scat, 0)
        return 0
    lax.fori_loop(0, _NSTRIP, p2, 0)

    def p3(j, w):
        p = pos[pl.ds(j * 16, 16)]
        m = p >= 0
        node = base + j * 16 + iota
        plsc.store_compressed(wn1.at[pl.ds(w, 16)], node, mask=m)
        plsc.store_compressed(wp1.at[pl.ds(w, 16)], p, mask=m)
        return w + jnp.sum(m.astype(jnp.int32))
    w_cnt = lax.fori_loop(0, _PCAP // 16, p3, jnp.int32(0))

    n_wchunk = (w_cnt + _CHK - 1) // _CHK

    @pl.when(w_cnt > 0)
    def _():
        z16 = jnp.zeros((16,), jnp.int32)
        padn = plsc.load_gather(wn1, [z16])
        padp = plsc.load_gather(wp1, [z16])

        def pad(j, _):
            mval = (j * 16 + iota) < w_cnt
            wn1[pl.ds(j * 16, 16)] = jnp.where(mval, wn1[pl.ds(j * 16, 16)],
                                               padn)
            wp1[pl.ds(j * 16, 16)] = jnp.where(mval, wp1[pl.ds(j * 16, 16)],
                                               padp)
            return 0
        lax.fori_loop(w_cnt // 16, n_wchunk * (_CHK // 16), pad, 0)

        def tocol(j, _):
            wn2[j // 8, pl.ds((j % 8) * 16, 16)] = wn1[pl.ds(j * 16, 16)]
            return 0
        lax.fori_loop(0, n_wchunk * (_CHK // 16), tocol, 0)

    def _stage(cur, other, k):
        @pl.when(k >= 1)
        def _():
            pltpu.make_async_copy(
                other, newmem_hbm.at[pl.ds(base + (k - 1) * _CHK, _CHK)],
                sems).wait()

        @pl.when(k + 1 < nch)
        def _():
            pltpu.async_copy(mem_hbm.at[pl.ds(base + (k + 1) * _CHK, _CHK)],
                             other, seml)
        pltpu.make_async_copy(mem_hbm.at[pl.ds(base + k * _CHK, _CHK)], cur,
                              seml).wait()
        pltpu.async_copy(cur, newmem_hbm.at[pl.ds(base + k * _CHK, _CHK)],
                         sems)

    def copy_body(k, _):
        @pl.when(k % 2 == 0)
        def _():
            _stage(bufa, bufb, k)

        @pl.when(k % 2 == 1)
        def _():
            _stage(bufb, bufa, k)
        return 0
    lax.fori_loop(0, nch, copy_body, 0)
    pltpu.make_async_copy(bufa, newmem_hbm.at[pl.ds(base, _CHK)], sems).wait()

    @pl.when(wid == _NW - 1)
    def _():
        lb = (_NW - 1) * _OWN + (_LAST // _CHK) * _CHK
        pltpu.sync_copy(mem_hbm.at[pl.ds(lb, N - lb)],
                        bufa.at[pl.ds(0, N - lb)])
        pltpu.sync_copy(bufa.at[pl.ds(0, N - lb)],
                        newmem_hbm.at[pl.ds(lb, N - lb)])

    @pl.when(wid < _NW - 1)
    def _():
        pltpu.sync_copy(lu_hbm.at[pl.ds(base, _OWN)], luv)
        pltpu.sync_copy(luv, newlu_hbm.at[pl.ds(base, _OWN)])

    @pl.when(wid == _NW - 1)
    def _():
        lb = (_NW - 1) * _OWN
        pltpu.sync_copy(lu_hbm.at[pl.ds(lb, _LAST)], luv.at[pl.ds(0, _LAST)])
        pltpu.sync_copy(luv.at[pl.ds(0, _LAST)], newlu_hbm.at[pl.ds(lb, _LAST)])

    def winner_chunk(c2, _):
        pltpu.async_copy(upd_hbm.at[wp1.at[pl.ds(c2 * _CHK, _CHK)]], wrows,
                         sem).wait()
        pltpu.sync_copy(wrows, newmem_hbm.at[wn2.at[c2]])
        pltpu.async_copy(t_hbm.at[wp1.at[pl.ds(c2 * _CHK, _CHK)]], twv,
                         sem).wait()
        pltpu.sync_copy(twv, newlu_hbm.at[wn2.at[c2]])
        return 0
    lax.fori_loop(0, n_wchunk, winner_chunk, 0)


_sc_update = pl.kernel(
    _update_body,
    out_type=[
        jax.ShapeDtypeStruct((N, D_MEM), jnp.float32),
        jax.ShapeDtypeStruct((N,), jnp.float32),
    ],
    mesh=plsc.VectorSubcoreMesh(core_axis_name="c", subcore_axis_name="s"),
    scratch_types=[
        pltpu.VMEM((_STRIP,), jnp.int32),
        pltpu.VMEM((_STRIP + 16,), jnp.int32),
        pltpu.VMEM((_STRIP + 16,), jnp.int32),
        pltpu.VMEM((_PCAP,), jnp.int32),
        pltpu.VMEM((_PCAP + 16,), jnp.int32),
        pltpu.VMEM((_PCAP + 16,), jnp.int32),
        pltpu.VMEM((_PCAP // _CHK, _CHK), jnp.int32),
        pltpu.VMEM((_CHK, D_MEM), jnp.float32),
        pltpu.VMEM((_CHK, D_MEM), jnp.float32),
        pltpu.VMEM((_CHK, D_MEM), jnp.float32),
        pltpu.VMEM((_OWN,), jnp.float32),
        pltpu.VMEM((_CHK,), jnp.float32),
        pltpu.SemaphoreType.DMA,
        pltpu.SemaphoreType.DMA,
        pltpu.SemaphoreType.DMA,
    ],
    compiler_params=pltpu.CompilerParams(needs_layout_passes=False),
)


def kernel(edge_index, t, msg, memory, last_update, w_t, b_t, W_nbr, W_self, W_upd):
    src, dst = edge_index[0], edge_index[1]
    mem_src, mem_dst, lu_src = _sc_gather(memory, edge_index, last_update)
    t2 = t[:, None]
    lu2 = lu_src[:, None]
    m, upd = _dense(
        mem_src, mem_dst, msg, t2, lu2, w_t, b_t.reshape(1, D_TIME),
        W_nbr, W_upd)
    agg_src, agg_dst = _sc_agg(edge_index, m)
    new_memory, new_last_update = _sc_update(
        memory, edge_index, t, last_update, upd)
    h_src, h_dst = _head(mem_src, mem_dst, agg_src, agg_dst, W_self)
    return (h_src, h_dst, new_memory, new_last_update)


# speedup vs baseline: 3.1231x; 1.0002x over previous
import jax
import jax.numpy as jnp
from jax import lax
from jax.experimental import pallas as pl
from jax.experimental.pallas import tpu as pltpu
from jax.experimental.pallas import tpu_sc as plsc

N = 100000
B = 32768
D_MEM = 128
D_MSG = 16
D_TIME = 32
D_OUT = 128

_BLK = 2048


def _dense_body(mem_src_ref, mem_dst_ref, msg_ref, t_ref, lu_src_ref,
                w_t_ref, b_t_ref, W_nbr_ref, W_upd_ref,
                m_ref, upd_ref):
    mem_src = mem_src_ref[...]
    mem_dst = mem_dst_ref[...]
    msg = msg_ref[...]
    rel = lu_src_ref[...] - t_ref[...]
    w_t = w_t_ref[...]
    b_t = b_t_ref[...]
    enc_rel = jnp.cos(rel * w_t + b_t)
    enc_t = jnp.cos((-rel) * w_t + b_t)
    nbr_in = jnp.concatenate([mem_src, msg, enc_rel], axis=1)
    m_ref[...] = jax.nn.relu(
        jax.lax.dot(nbr_in, W_nbr_ref[...],
                    preferred_element_type=jnp.float32))
    upd_in = jnp.concatenate([mem_src, mem_dst, msg, enc_t], axis=1)
    upd_ref[...] = jnp.tanh(
        jax.lax.dot(upd_in, W_upd_ref[...],
                    preferred_element_type=jnp.float32))


def _dense(mem_src, mem_dst, msg, t, lu_src, w_t, b_t, W_nbr, W_upd):
    grid = (B // _BLK,)
    row_spec = lambda d: pl.BlockSpec((_BLK, d), lambda i: (i, 0))
    full = lambda a, b: pl.BlockSpec((a, b), lambda i: (0, 0))
    return pl.pallas_call(
        _dense_body,
        grid=grid,
        in_specs=[
            row_spec(D_MEM), row_spec(D_MEM), row_spec(D_MSG),
            row_spec(1), row_spec(1),
            full(1, D_TIME), full(1, D_TIME),
            full(D_MEM + D_MSG + D_TIME, D_OUT),
            full(2 * D_MEM + D_MSG + D_TIME, D_MEM),
        ],
        out_specs=[row_spec(D_OUT), row_spec(D_MEM)],
        out_shape=[
            jax.ShapeDtypeStruct((B, D_OUT), jnp.float32),
            jax.ShapeDtypeStruct((B, D_MEM), jnp.float32),
        ],
    )(mem_src, mem_dst, msg, t, lu_src, w_t, b_t, W_nbr, W_upd)


def _head_body(mem_src_ref, mem_dst_ref, as_ref, ad_ref, W_self_ref,
               hs_ref, hd_ref):
    W_self = W_self_ref[...]
    hs_ref[...] = jax.nn.relu(
        jax.lax.dot(mem_src_ref[...], W_self,
                    preferred_element_type=jnp.float32) + as_ref[...])
    hd_ref[...] = jax.nn.relu(
        jax.lax.dot(mem_dst_ref[...], W_self,
                    preferred_element_type=jnp.float32) + ad_ref[...])


def _head(mem_src, mem_dst, agg_src, agg_dst, W_self):
    grid = (B // _BLK,)
    row_spec = lambda d: pl.BlockSpec((_BLK, d), lambda i: (i, 0))
    full = lambda a, b: pl.BlockSpec((a, b), lambda i: (0, 0))
    return pl.pallas_call(
        _head_body,
        grid=grid,
        in_specs=[row_spec(D_MEM), row_spec(D_MEM),
                  row_spec(D_OUT), row_spec(D_OUT),
                  full(D_MEM, D_OUT)],
        out_specs=[row_spec(D_OUT), row_spec(D_OUT)],
        out_shape=[
            jax.ShapeDtypeStruct((B, D_OUT), jnp.float32),
            jax.ShapeDtypeStruct((B, D_OUT), jnp.float32),
        ],
    )(mem_src, mem_dst, agg_src, agg_dst, W_self)


_INFO = plsc.get_sparse_core_info()
_NC, _NS = _INFO.num_cores, _INFO.num_subcores
_NW = _NC * _NS
_EV_W = B // _NW
_CH = 512


def _gather_body(mem_hbm, ei_hbm, lu_hbm, ms_out, md_out, lus_out,
                 idx_v, rows_v, lu_v, sem):
    wid = lax.axis_index("s") * _NC + lax.axis_index("c")
    base = wid * _EV_W
    pltpu.sync_copy(ei_hbm.at[0, pl.ds(base, _EV_W)], idx_v)
    pltpu.async_copy(lu_hbm.at[idx_v], lu_v, sem).wait()
    pltpu.sync_copy(lu_v, lus_out.at[pl.ds(base, _EV_W)])
    for half in range(_EV_W // _CH):
        off = half * _CH
        pltpu.async_copy(mem_hbm.at[idx_v.at[pl.ds(off, _CH)]], rows_v,
                         sem).wait()
        pltpu.sync_copy(rows_v, ms_out.at[pl.ds(base + off, _CH)])
    pltpu.sync_copy(ei_hbm.at[1, pl.ds(base, _EV_W)], idx_v)
    for half in range(_EV_W // _CH):
        off = half * _CH
        pltpu.async_copy(mem_hbm.at[idx_v.at[pl.ds(off, _CH)]], rows_v,
                         sem).wait()
        pltpu.sync_copy(rows_v, md_out.at[pl.ds(base + off, _CH)])


_sc_gather = pl.kernel(
    _gather_body,
    out_type=[
        jax.ShapeDtypeStruct((B, D_MEM), jnp.float32),
        jax.ShapeDtypeStruct((B, D_MEM), jnp.float32),
        jax.ShapeDtypeStruct((B,), jnp.float32),
    ],
    mesh=plsc.VectorSubcoreMesh(core_axis_name="c", subcore_axis_name="s"),
    scratch_types=[
        pltpu.VMEM((_EV_W,), jnp.int32),
        pltpu.VMEM((_CH, D_MEM), jnp.float32),
        pltpu.VMEM((_EV_W,), jnp.float32),
        pltpu.SemaphoreType.DMA,
    ],
)


_AGG_R = 13055
_AGG_PASSES = (N + 2 * _AGG_R - 1) // (2 * _AGG_R)
_EV_T = B // _NS
_ACH = 64
_CROWS = (_EV_T + _ACH - 1) // _ACH + 1
_ZSH = (_AGG_R + 1) // _NS
_ZCH = 48


def _agg_compact(ev_ref, base, iota, cv, cb, sid):
    def compact(j, off):
        v = ev_ref[pl.ds(j * 16, 16)]
        rel = v - base
        m_in = (rel >= 0) & (rel < _AGG_R)
        bv = sid * _EV_T + j * 16 + iota
        plsc.store_compressed(cv.at[pl.ds(off, 16)], rel, mask=m_in)
        plsc.store_compressed(cb.at[pl.ds(off, 16)], bv, mask=m_in)
        return off + jnp.sum(m_in.astype(jnp.int32))
    return lax.fori_loop(0, _EV_T // 16, compact, jnp.int32(0))


def _agg_pad(cv, cb, off, iota, padv, padb):
    nch = (off + _ACH - 1) // _ACH

    def pad(j, _):
        mval = (j * 16 + iota) < off
        cv[pl.ds(j * 16, 16)] = jnp.where(mval, cv[pl.ds(j * 16, 16)], padv)
        cb[pl.ds(j * 16, 16)] = jnp.where(mval, cb[pl.ds(j * 16, 16)], padb)
        return 0
    lax.fori_loop(off // 16, nch * (_ACH // 16), pad, 0)
    return nch


def _agg_tocol(cv, cv2, nch):
    vpr = _ACH // 16

    def tocol(j, _):
        cv2[j // vpr, pl.ds((j % vpr) * 16, 16)] = cv[pl.ds(j * 16, 16)]
        return 0
    lax.fori_loop(0, nch * vpr, tocol, 0)


def _agg_body(ei_hbm, m_hbm, as_hbm, ad_hbm,
              ev_src, ev_dst, cvd, cbd, cvs, cbs, cv2, rows,
              shared, sem):
    sid = lax.axis_index("s")
    core = lax.axis_index("c")
    iota = lax.iota(jnp.int32, 16)
    z16 = jnp.zeros((16,), jnp.float32)
    pltpu.sync_copy(ei_hbm.at[0, pl.ds(sid * _EV_T, _EV_T)], ev_src)
    pltpu.sync_copy(ei_hbm.at[1, pl.ds(sid * _EV_T, _EV_T)], ev_dst)

    def one_pass(r, _):
        base = r * (2 * _AGG_R) + core * _AGG_R

        def zb(i, _):
            rows[i // 8, pl.ds((i % 8) * 16, 16)] = z16
            return 0
        lax.fori_loop(0, _ACH * 8, zb, 0)

        def zero(q, _):
            pltpu.sync_copy(rows.at[pl.ds(0, _ZCH)],
                            shared.at[pl.ds(sid * _ZSH + q * _ZCH, _ZCH)])
            return 0
        lax.fori_loop(0, _ZSH // _ZCH, zero, 0)
        plsc.subcore_barrier()

        offd = _agg_compact(ev_dst, base, iota, cvd, cbd, sid)
        nchd = _agg_pad(cvd, cbd, offd, iota,
                        jnp.zeros((16,), jnp.int32) + _AGG_R,
                        jnp.zeros((16,), jnp.int32))
        _agg_tocol(cvd, cv2, nchd)

        def add_chunk(c, _):
            pltpu.async_copy(m_hbm.at[cbd.at[pl.ds(c * _ACH, _ACH)]], rows,
                             sem).wait()
            pltpu.sync_copy(rows, shared.at[cv2.at[c]], add=True)
            return 0
        lax.fori_loop(0, nchd, add_chunk, 0)
        plsc.subcore_barrier()

        def emit(cv, cb, off, out_hbm):
            @pl.when(off > 0)
            def _():
                lastv = plsc.load_gather(cv, [iota * 0 + off - 1])
                lastb = plsc.load_gather(cb, [iota * 0 + off - 1])
                nch = _agg_pad(cv, cb, off, iota, lastv, lastb)
                _agg_tocol(cb, cv2, nch)

                def out_chunk(c, _):
                    pltpu.async_copy(
                        shared.at[cv.at[pl.ds(c * _ACH, _ACH)]], rows,
                        sem).wait()
                    pltpu.sync_copy(rows, out_hbm.at[cv2.at[c]])
                    return 0
                lax.fori_loop(0, nch, out_chunk, 0)

        offs = _agg_compact(ev_src, base, iota, cvs, cbs, sid)
        emit(cvs, cbs, offs, as_hbm)
        emit(cvd, cbd, offd, ad_hbm)
        plsc.subcore_barrier()
        return 0
    lax.fori_loop(0, _AGG_PASSES, one_pass, 0)


_sc_agg = pl.kernel(
    _agg_body,
    out_type=[
        jax.ShapeDtypeStruct((B, D_OUT), jnp.float32),
        jax.ShapeDtypeStruct((B, D_OUT), jnp.float32),
    ],
    mesh=plsc.VectorSubcoreMesh(core_axis_name="c", subcore_axis_name="s"),
    scratch_types=[
        pltpu.VMEM((_EV_T,), jnp.int32),
        pltpu.VMEM((_EV_T,), jnp.int32),
        pltpu.VMEM((_EV_T + 16,), jnp.int32),
        pltpu.VMEM((_EV_T + 16,), jnp.int32),
        pltpu.VMEM((_EV_T + 16,), jnp.int32),
        pltpu.VMEM((_EV_T + 16,), jnp.int32),
        pltpu.VMEM((_CROWS, _ACH), jnp.int32),
        pltpu.VMEM((_ACH, D_OUT), jnp.float32),
        pltpu.VMEM_SHARED((_AGG_R + 1, D_OUT), jnp.float32),
        pltpu.SemaphoreType.DMA,
    ],
    compiler_params=pltpu.CompilerParams(needs_layout_passes=False),
)


_CHK = 128
_OWN = 3200
_LAST = N - (_NW - 1) * _OWN
_PCAP = _OWN
_STRIP = 2048
_NSTRIP = B // _STRIP


def _update_body(mem_hbm, ei_hbm, t_hbm, lu_hbm, upd_hbm,
                 newmem_hbm, newlu_hbm,
                 ss, cv, cb, pos, wn1, wp1, wn2, wrows, bufa, bufb, luv, twv,
                 sem, seml, sems):
    wid = lax.axis_index("s") * _NC + lax.axis_index("c")
    base = wid * _OWN
    iota = lax.iota(jnp.int32, 16)
    neg1 = jnp.zeros((16,), jnp.int32) - 1

    nch = (jnp.minimum(jnp.int32(_OWN), jnp.int32(N) - base)) // _CHK
    pltpu.async_copy(mem_hbm.at[pl.ds(base, _CHK)], bufa, seml)

    def p1(i, _):
        pos[pl.ds(i * 16, 16)] = neg1
        return 0
    lax.fori_loop(0, _PCAP // 16, p1, 0)

    def p2(s, _):
        pltpu.sync_copy(ei_hbm.at[0, pl.ds(s * _STRIP, _STRIP)], ss)

        def compact(j, off):
            v = ss[pl.ds(j * 16, 16)]
            slot = v - base
            own = (slot >= 0) & (slot < _OWN)
            bv = s * _STRIP + j * 16 + iota
            plsc.store_compressed(cv.at[pl.ds(off, 16)], slot, mask=own)
            plsc.store_compressed(cb.at[pl.ds(off, 16)], bv, mask=own)
            return off + jnp.sum(own.astype(jnp.int32))
        off = lax.fori_loop(0, _STRIP // 16, compact, jnp.int32(0))

        def scat(k, _):
            sl = cv[pl.ds(k * 16, 16)]
            bvv = cb[pl.ds(k * 16, 16)]
            valid = (k * 16 + iota) < off
            _, lastm = plsc.scan_count(sl, valid)
            plsc.store_scatter(pos, [sl], bvv, mask=lastm & valid)
            return 0
        lax.fori_loop(0, (off + 15) // 16, scat, 0)
        return 0
    lax.fori_loop(0, _NSTRIP, p2, 0)

    def p3(j, w):
        p = pos[pl.ds(j * 16, 16)]
        m = p >= 0
        node = base + j * 16 + iota
        plsc.store_compressed(wn1.at[pl.ds(w, 16)], node, mask=m)
        plsc.store_compressed(wp1.at[pl.ds(w, 16)], p, mask=m)
        return w + jnp.sum(m.astype(jnp.int32))
    w_cnt = lax.fori_loop(0, _PCAP // 16, p3, jnp.int32(0))

    n_wchunk = (w_cnt + _CHK - 1) // _CHK

    @pl.when(w_cnt > 0)
    def _():
        z16 = jnp.zeros((16,), jnp.int32)
        padn = plsc.load_gather(wn1, [z16])
        padp = plsc.load_gather(wp1, [z16])

        def pad(j, _):
            mval = (j * 16 + iota) < w_cnt
            wn1[pl.ds(j * 16, 16)] = jnp.where(mval, wn1[pl.ds(j * 16, 16)],
                                               padn)
            wp1[pl.ds(j * 16, 16)] = jnp.where(mval, wp1[pl.ds(j * 16, 16)],
                                               padp)
            return 0
        lax.fori_loop(w_cnt // 16, n_wchunk * (_CHK // 16), pad, 0)

        def tocol(j, _):
            wn2[j // 8, pl.ds((j % 8) * 16, 16)] = wn1[pl.ds(j * 16, 16)]
            return 0
        lax.fori_loop(0, n_wchunk * (_CHK // 16), tocol, 0)

    def _stage(cur, other, k):
        @pl.when(k >= 1)
        def _():
            pltpu.make_async_copy(
                other, newmem_hbm.at[pl.ds(base + (k - 1) * _CHK, _CHK)],
                sems).wait()

        @pl.when(k + 1 < nch)
        def _():
            pltpu.async_copy(mem_hbm.at[pl.ds(base + (k + 1) * _CHK, _CHK)],
                             other, seml)
        pltpu.make_async_copy(mem_hbm.at[pl.ds(base + k * _CHK, _CHK)], cur,
                              seml).wait()
        pltpu.async_copy(cur, newmem_hbm.at[pl.ds(base + k * _CHK, _CHK)],
                         sems)

    def copy_body(k, _):
        @pl.when(k % 2 == 0)
        def _():
            _stage(bufa, bufb, k)

        @pl.when(k % 2 == 1)
        def _():
            _stage(bufb, bufa, k)
        return 0
    lax.fori_loop(0, nch, copy_body, 0)
    pltpu.make_async_copy(bufa, newmem_hbm.at[pl.ds(base, _CHK)], sems).wait()

    @pl.when(wid == _NW - 1)
    def _():
        lb = (_NW - 1) * _OWN + (_LAST // _CHK) * _CHK
        pltpu.sync_copy(mem_hbm.at[pl.ds(lb, N - lb)],
                        bufa.at[pl.ds(0, N - lb)])
        pltpu.sync_copy(bufa.at[pl.ds(0, N - lb)],
                        newmem_hbm.at[pl.ds(lb, N - lb)])

    @pl.when(wid < _NW - 1)
    def _():
        pltpu.sync_copy(lu_hbm.at[pl.ds(base, _OWN)], luv)
        pltpu.sync_copy(luv, newlu_hbm.at[pl.ds(base, _OWN)])

    @pl.when(wid == _NW - 1)
    def _():
        lb = (_NW - 1) * _OWN
        pltpu.sync_copy(lu_hbm.at[pl.ds(lb, _LAST)], luv.at[pl.ds(0, _LAST)])
        pltpu.sync_copy(luv.at[pl.ds(0, _LAST)], newlu_hbm.at[pl.ds(lb, _LAST)])

    def winner_chunk(c2, _):
        pltpu.async_copy(upd_hbm.at[wp1.at[pl.ds(c2 * _CHK, _CHK)]], wrows,
                         sem).wait()
        pltpu.sync_copy(wrows, newmem_hbm.at[wn2.at[c2]])
        pltpu.async_copy(t_hbm.at[wp1.at[pl.ds(c2 * _CHK, _CHK)]], twv,
                         sem).wait()
        pltpu.sync_copy(twv, newlu_hbm.at[wn2.at[c2]])
        return 0
    lax.fori_loop(0, n_wchunk, winner_chunk, 0)


_sc_update = pl.kernel(
    _update_body,
    out_type=[
        jax.ShapeDtypeStruct((N, D_MEM), jnp.float32),
        jax.ShapeDtypeStruct((N,), jnp.float32),
    ],
    mesh=plsc.VectorSubcoreMesh(core_axis_name="c", subcore_axis_name="s"),
    scratch_types=[
        pltpu.VMEM((_STRIP,), jnp.int32),
        pltpu.VMEM((_STRIP + 16,), jnp.int32),
        pltpu.VMEM((_STRIP + 16,), jnp.int32),
        pltpu.VMEM((_PCAP,), jnp.int32),
        pltpu.VMEM((_PCAP + 16,), jnp.int32),
        pltpu.VMEM((_PCAP + 16,), jnp.int32),
        pltpu.VMEM((_PCAP // _CHK, _CHK), jnp.int32),
        pltpu.VMEM((_CHK, D_MEM), jnp.float32),
        pltpu.VMEM((_CHK, D_MEM), jnp.float32),
        pltpu.VMEM((_CHK, D_MEM), jnp.float32),
        pltpu.VMEM((_OWN,), jnp.float32),
        pltpu.VMEM((_CHK,), jnp.float32),
        pltpu.SemaphoreType.DMA,
        pltpu.SemaphoreType.DMA,
        pltpu.SemaphoreType.DMA,
    ],
    compiler_params=pltpu.CompilerParams(needs_layout_passes=False),
)


def kernel(edge_index, t, msg, memory, last_update, w_t, b_t, W_nbr, W_self, W_upd):
    src, dst = edge_index[0], edge_index[1]
    mem_src, mem_dst, lu_src = _sc_gather(memory, edge_index, last_update)
    t2 = t[:, None]
    lu2 = lu_src[:, None]
    m, upd = _dense(
        mem_src, mem_dst, msg, t2, lu2, w_t, b_t.reshape(1, D_TIME),
        W_nbr, W_upd)
    agg_src, agg_dst = _sc_agg(edge_index, m)
    new_memory, new_last_update = _sc_update(
        memory, edge_index, t, last_update, upd)
    h_src, h_dst = _head(mem_src, mem_dst, agg_src, agg_dst, W_self)
    return (h_src, h_dst, new_memory, new_last_update)
